# Initial kernel scaffold; baseline (speedup 1.0000x reference)
import jax
import jax.numpy as jnp
from jax.experimental import pallas as pl


def kernel(x, edge_index, W1, b1, W2, b2, W3, b3):
    raise NotImplementedError("write your pallas kernel here")


# speedup vs baseline: 21.6672x; 21.6672x over previous
import functools

import jax
import jax.numpy as jnp
from jax import lax
from jax.experimental import pallas as pl
from jax.experimental.pallas import tpu as pltpu
from jax.experimental.pallas import tpu_sc as plsc

N = 10000
E = 320000
N_ACC = 10240
ROWS_PER_SUB = N_ACC // 16

F32 = jnp.float32


def _sc_degree(sidx):
    mesh = plsc.VectorSubcoreMesh(core_axis_name="c", subcore_axis_name="s")

    @functools.partial(
        pl.kernel,
        out_type=jax.ShapeDtypeStruct((2, N_ACC), F32),
        mesh=mesh,
        scratch_types=[
            pltpu.VMEM((16, 128), jnp.int32),
            pltpu.VMEM((ROWS_PER_SUB,), F32),
            pltpu.VMEM((128,), F32),
            pltpu.VMEM_SHARED((N_ACC,), F32),
        ],
    )
    def k(sidx_h, out_h, sbuf, zvec, ones, hist):
        c = lax.axis_index("c")
        s = lax.axis_index("s")

        @pl.loop(0, ROWS_PER_SUB // 16)
        def _(i):
            zvec[pl.ds(i * 16, 16)] = jnp.zeros((16,), F32)

        @pl.loop(0, 8)
        def _(i):
            ones[pl.ds(i * 16, 16)] = jnp.ones((16,), F32)

        pltpu.sync_copy(zvec, hist.at[pl.ds(s * ROWS_PER_SUB, ROWS_PER_SUB)])
        plsc.subcore_barrier()

        @pl.loop(0, 5)
        def _(sb):
            pltpu.sync_copy(sidx_h.at[c, pl.ds(s * 80 + sb * 16, 16)], sbuf)
            for j in range(16):
                pltpu.sync_copy(ones, hist.at[sbuf.at[j]], add=True)

        plsc.subcore_barrier()
        pltpu.sync_copy(hist.at[pl.ds(s * ROWS_PER_SUB, ROWS_PER_SUB)],
                        out_h.at[c, pl.ds(s * ROWS_PER_SUB, ROWS_PER_SUB)])

    return k(sidx)


def _sc_aggregate(table, gidx, sidx, rows_per_worker):
    n_sb = rows_per_worker // 16
    mesh = plsc.VectorSubcoreMesh(core_axis_name="c", subcore_axis_name="s")

    @functools.partial(
        pl.kernel,
        out_type=jax.ShapeDtypeStruct((2, N_ACC, 128), F32),
        mesh=mesh,
        scratch_types=[
            pltpu.VMEM((16, 128), jnp.int32),
            pltpu.VMEM((16, 128), jnp.int32),
            pltpu.VMEM((128, 128), F32),
            pltpu.VMEM((128, 128), F32),
            pltpu.VMEM((64, 128), F32),
            pltpu.VMEM_SHARED((N_ACC, 128), F32),
            pltpu.SemaphoreType.DMA,
            pltpu.SemaphoreType.DMA,
        ],
    )
    def k(table_h, gidx_h, sidx_h, out_h,
          gbuf, sbuf, rbufa, rbufb, zbuf, acc, sema, semb):
        c = lax.axis_index("c")
        s = lax.axis_index("s")

        @pl.loop(0, 64)
        def _(i):
            @pl.loop(0, 8)
            def _(kk):
                zbuf[i, pl.ds(kk * 16, 16)] = jnp.zeros((16,), F32)

        @pl.loop(0, ROWS_PER_SUB // 64)
        def _(i):
            pltpu.sync_copy(zbuf, acc.at[pl.ds(s * ROWS_PER_SUB + i * 64, 64)])
        plsc.subcore_barrier()

        base = s * rows_per_worker

        @pl.loop(0, n_sb)
        def _(sb):
            r0 = base + sb * 16
            pltpu.sync_copy(gidx_h.at[c, pl.ds(r0, 16)], gbuf)
            pltpu.sync_copy(sidx_h.at[c, pl.ds(r0, 16)], sbuf)
            cpa = pltpu.async_copy(table_h.at[gbuf.at[0]], rbufa, sema)
            for j in range(16):
                buf_cur, buf_nxt = (rbufa, rbufb) if j % 2 == 0 else (rbufb, rbufa)
                sem_cur, sem_nxt = (sema, semb) if j % 2 == 0 else (semb, sema)
                if j < 15:
                    nxt = pltpu.async_copy(table_h.at[gbuf.at[j + 1]], buf_nxt,
                                           sem_nxt)
                pltpu.make_async_copy(table_h.at[gbuf.at[j]], buf_cur,
                                      sem_cur).wait()
                pltpu.sync_copy(buf_cur, acc.at[sbuf.at[j]], add=True)

        plsc.subcore_barrier()
        pltpu.sync_copy(acc.at[pl.ds(s * ROWS_PER_SUB, ROWS_PER_SUB)],
                        out_h.at[c, pl.ds(s * ROWS_PER_SUB, ROWS_PER_SUB)])

    return k(table, gidx, sidx)


_RB = 2000


def _tc_mm1(x, w1):
    def body(x_ref, w_ref, o_ref):
        o_ref[...] = jnp.dot(x_ref[...], w_ref[...],
                             preferred_element_type=F32)

    return pl.pallas_call(
        body,
        grid=(N // _RB,),
        in_specs=[
            pl.BlockSpec((_RB, 128), lambda i: (i, 0)),
            pl.BlockSpec((128, 256), lambda i: (0, 0)),
        ],
        out_specs=pl.BlockSpec((_RB, 256), lambda i: (i, 0)),
        out_shape=jax.ShapeDtypeStruct((N, 256), F32),
    )(x, w1)


def _tc_dinv(deg2):
    def body(d_ref, o_ref):
        deg = d_ref[0] + d_ref[1] + 1.0
        o_ref[...] = lax.rsqrt(deg)

    return pl.pallas_call(
        body,
        in_specs=[pl.BlockSpec((2, 80, 128), lambda: (0, 0, 0))],
        out_specs=pl.BlockSpec((80, 128), lambda: (0, 0)),
        out_shape=jax.ShapeDtypeStruct((80, 128), F32),
    )(deg2)


def _tc_scale_split(hw, dinv_col):
    def body(h_ref, dv_ref, o_ref):
        dv = dv_ref[...]
        o_ref[0] = h_ref[:, :128] * dv
        o_ref[1] = h_ref[:, 128:] * dv

    return pl.pallas_call(
        body,
        grid=(N // _RB,),
        in_specs=[
            pl.BlockSpec((_RB, 256), lambda i: (i, 0)),
            pl.BlockSpec((_RB, 1), lambda i: (i, 0)),
        ],
        out_specs=pl.BlockSpec((2, _RB, 128), lambda i: (0, i, 0)),
        out_shape=jax.ShapeDtypeStruct((2, N, 128), F32),
    )(hw, dinv_col)


def _tc_mid(acc, u, dinv_col, b2d, w, out_c):
    def body(a_ref, u_ref, dv_ref, b_ref, w_ref, o_ref):
        dv = dv_ref[...]
        h0 = jnp.maximum((a_ref[0] + u_ref[0]) * dv + b_ref[:, :128], 0.0)
        h1 = jnp.maximum((a_ref[1] + u_ref[1]) * dv + b_ref[:, 128:], 0.0)
        hw = (jnp.dot(h0, w_ref[:128], preferred_element_type=F32)
              + jnp.dot(h1, w_ref[128:], preferred_element_type=F32))
        if out_c == 256:
            o_ref[0] = hw[:, :128] * dv
            o_ref[1] = hw[:, 128:] * dv
        else:
            o_ref[...] = hw * dv

    if out_c == 256:
        out_spec = pl.BlockSpec((2, _RB, 128), lambda i: (0, i, 0))
        out_shape = jax.ShapeDtypeStruct((2, N, 128), F32)
    else:
        out_spec = pl.BlockSpec((_RB, 128), lambda i: (i, 0))
        out_shape = jax.ShapeDtypeStruct((N, 128), F32)

    return pl.pallas_call(
        body,
        grid=(N // _RB,),
        in_specs=[
            pl.BlockSpec((2, _RB, 128), lambda i: (0, i, 0)),
            pl.BlockSpec((2, _RB, 128), lambda i: (0, i, 0)),
            pl.BlockSpec((_RB, 1), lambda i: (i, 0)),
            pl.BlockSpec((1, 256), lambda i: (0, 0)),
            pl.BlockSpec((256, out_c), lambda i: (0, 0)),
        ],
        out_specs=out_spec,
        out_shape=out_shape,
    )(acc, u, dinv_col, b2d, w)


def _tc_final(acc3, u3, dinv_col, b2d):
    def body(a_ref, u_ref, dv_ref, b_ref, o_ref):
        o_ref[...] = ((a_ref[0] + a_ref[1] + u_ref[...]) * dv_ref[...]
                      + b_ref[...])

    return pl.pallas_call(
        body,
        grid=(N // _RB,),
        in_specs=[
            pl.BlockSpec((2, _RB, 128), lambda i: (0, i, 0)),
            pl.BlockSpec((_RB, 128), lambda i: (i, 0)),
            pl.BlockSpec((_RB, 1), lambda i: (i, 0)),
            pl.BlockSpec((1, 128), lambda i: (0, 0)),
        ],
        out_specs=pl.BlockSpec((_RB, 128), lambda i: (i, 0)),
        out_shape=jax.ShapeDtypeStruct((N, 128), F32),
    )(acc3, u3, dinv_col, b2d)


def _prep_indices(edge_index):
    src = edge_index[0]
    dst = edge_index[1]
    w = jnp.arange(16, dtype=jnp.int32)[:, None]

    pad12 = 480
    padg = (w * 997 + jnp.arange(pad12, dtype=jnp.int32)[None, :] * 13) % N
    pads = N + (w * 31 + jnp.arange(pad12, dtype=jnp.int32)[None, :]) % (N_ACC - N)
    sw = jnp.concatenate([src.reshape(16, E // 16), padg], axis=1)
    dw = jnp.concatenate([dst.reshape(16, E // 16), pads], axis=1)
    gidx12 = jnp.stack([sw, sw + N]).reshape(2, 2560, 128)
    sidx12 = jnp.stack([dw, dw]).reshape(2, 2560, 128)

    pad3 = 240
    padg3 = (w * 997 + jnp.arange(pad3, dtype=jnp.int32)[None, :] * 13) % N
    pads3 = N + (w * 31 + jnp.arange(pad3, dtype=jnp.int32)[None, :]) % (N_ACC - N)
    s3 = src.reshape(2, 16, E // 32)
    d3 = dst.reshape(2, 16, E // 32)
    gidx3 = jnp.concatenate(
        [s3, jnp.broadcast_to(padg3, (2, 16, pad3))], axis=2).reshape(2, 1280, 128)
    sidx3 = jnp.concatenate(
        [d3, jnp.broadcast_to(pads3, (2, 16, pad3))], axis=2).reshape(2, 1280, 128)
    return gidx12, sidx12, gidx3, sidx3


def kernel(x, edge_index, W1, b1, W2, b2, W3, b3):
    gidx12, sidx12, gidx3, sidx3 = _prep_indices(edge_index)

    deg2 = _sc_degree(sidx3)
    hw1 = _tc_mm1(x, W1)

    dinvp = _tc_dinv(deg2.reshape(2, 80, 128))
    dinv_col = dinvp.reshape(N_ACC, 1)[:N]

    u1 = _tc_scale_split(hw1, dinv_col)
    acc1 = _sc_aggregate(u1.reshape(2 * N, 128), gidx12, sidx12, 160)
    u2 = _tc_mid(acc1, u1, dinv_col, b1.reshape(1, 256), W2, 256)
    acc2 = _sc_aggregate(u2.reshape(2 * N, 128), gidx12, sidx12, 160)
    u3 = _tc_mid(acc2, u2, dinv_col, b2.reshape(1, 256), W3, 128)
    acc3 = _sc_aggregate(u3, gidx3, sidx3, 80)
    return _tc_final(acc3, u3, dinv_col, b3.reshape(1, 128))


# speedup vs baseline: 26.1092x; 1.2050x over previous
import functools

import jax
import jax.numpy as jnp
from jax import lax
from jax.experimental import pallas as pl
from jax.experimental.pallas import tpu as pltpu
from jax.experimental.pallas import tpu_sc as plsc

N = 10000
E = 320000
N_ACC = 10240
ROWS_PER_SUB = N_ACC // 16

F32 = jnp.float32


def _sc_degree(sidx):
    mesh = plsc.VectorSubcoreMesh(core_axis_name="c", subcore_axis_name="s")

    @functools.partial(
        pl.kernel,
        out_type=jax.ShapeDtypeStruct((2, N_ACC), F32),
        mesh=mesh,
        scratch_types=[
            pltpu.VMEM((16, 128), jnp.int32),
            pltpu.VMEM((ROWS_PER_SUB,), F32),
            pltpu.VMEM((128,), F32),
            pltpu.VMEM_SHARED((N_ACC,), F32),
        ],
    )
    def k(sidx_h, out_h, sbuf, zvec, ones, hist):
        c = lax.axis_index("c")
        s = lax.axis_index("s")

        @pl.loop(0, ROWS_PER_SUB // 16)
        def _(i):
            zvec[pl.ds(i * 16, 16)] = jnp.zeros((16,), F32)

        @pl.loop(0, 8)
        def _(i):
            ones[pl.ds(i * 16, 16)] = jnp.ones((16,), F32)

        pltpu.sync_copy(zvec, hist.at[pl.ds(s * ROWS_PER_SUB, ROWS_PER_SUB)])
        plsc.subcore_barrier()

        @pl.loop(0, 5)
        def _(sb):
            pltpu.sync_copy(sidx_h.at[c, pl.ds(s * 80 + sb * 16, 16)], sbuf)
            for j in range(16):
                pltpu.sync_copy(ones, hist.at[sbuf.at[j]], add=True)

        plsc.subcore_barrier()
        pltpu.sync_copy(hist.at[pl.ds(s * ROWS_PER_SUB, ROWS_PER_SUB)],
                        out_h.at[c, pl.ds(s * ROWS_PER_SUB, ROWS_PER_SUB)])

    return k(sidx)


def _sc_aggregate(table, gidx, sidx, rows_per_worker):
    n_sb = rows_per_worker // 16
    mesh = plsc.VectorSubcoreMesh(core_axis_name="c", subcore_axis_name="s")

    @functools.partial(
        pl.kernel,
        out_type=jax.ShapeDtypeStruct((2, N_ACC, 128), F32),
        mesh=mesh,
        scratch_types=[
            pltpu.VMEM((16, 128), jnp.int32),
            pltpu.VMEM((16, 128), jnp.int32),
            pltpu.VMEM((128, 128), F32),
            pltpu.VMEM((128, 128), F32),
            pltpu.VMEM((64, 128), F32),
            pltpu.VMEM_SHARED((N_ACC, 128), F32),
            pltpu.SemaphoreType.DMA,
            pltpu.SemaphoreType.DMA,
        ],
    )
    def k(table_h, gidx_h, sidx_h, out_h,
          gbuf, sbuf, rbufa, rbufb, zbuf, acc, sema, semb):
        c = lax.axis_index("c")
        s = lax.axis_index("s")

        @pl.loop(0, 64)
        def _(i):
            @pl.loop(0, 8)
            def _(kk):
                zbuf[i, pl.ds(kk * 16, 16)] = jnp.zeros((16,), F32)

        @pl.loop(0, ROWS_PER_SUB // 64)
        def _(i):
            pltpu.sync_copy(zbuf, acc.at[pl.ds(s * ROWS_PER_SUB + i * 64, 64)])
        plsc.subcore_barrier()

        base = s * rows_per_worker

        @pl.loop(0, n_sb)
        def _(sb):
            r0 = base + sb * 16
            pltpu.sync_copy(gidx_h.at[c, pl.ds(r0, 16)], gbuf)
            pltpu.sync_copy(sidx_h.at[c, pl.ds(r0, 16)], sbuf)
            cpa = pltpu.async_copy(table_h.at[gbuf.at[0]], rbufa, sema)
            for j in range(16):
                buf_cur, buf_nxt = (rbufa, rbufb) if j % 2 == 0 else (rbufb, rbufa)
                sem_cur, sem_nxt = (sema, semb) if j % 2 == 0 else (semb, sema)
                if j < 15:
                    nxt = pltpu.async_copy(table_h.at[gbuf.at[j + 1]], buf_nxt,
                                           sem_nxt)
                pltpu.make_async_copy(table_h.at[gbuf.at[j]], buf_cur,
                                      sem_cur).wait()
                pltpu.sync_copy(buf_cur, acc.at[sbuf.at[j]], add=True)

        plsc.subcore_barrier()
        pltpu.sync_copy(acc.at[pl.ds(s * ROWS_PER_SUB, ROWS_PER_SUB)],
                        out_h.at[c, pl.ds(s * ROWS_PER_SUB, ROWS_PER_SUB)])

    return k(table, gidx, sidx)


_RB = 2000


def _tc_scale1(x, dinv_col):
    def body(x_ref, dv_ref, o_ref):
        o_ref[...] = x_ref[...] * dv_ref[...]

    return pl.pallas_call(
        body,
        grid=(N // _RB,),
        in_specs=[
            pl.BlockSpec((_RB, 128), lambda i: (i, 0)),
            pl.BlockSpec((_RB, 1), lambda i: (i, 0)),
        ],
        out_specs=pl.BlockSpec((_RB, 128), lambda i: (i, 0)),
        out_shape=jax.ShapeDtypeStruct((N, 128), F32),
    )(x, dinv_col)


def _tc_l12(acc0, v1, dinv_col, w1, b1_2d, w2):
    def body(a_ref, v_ref, dv_ref, w1_ref, b_ref, w2_ref, o_ref):
        dv = dv_ref[...]
        z1 = (a_ref[0] + a_ref[1] + v_ref[...]) * dv
        h2 = jnp.maximum(
            jnp.dot(z1, w1_ref[...], preferred_element_type=F32) + b_ref[...],
            0.0)
        hw = jnp.dot(h2, w2_ref[...], preferred_element_type=F32)
        o_ref[0] = hw[:, :128] * dv
        o_ref[1] = hw[:, 128:] * dv

    return pl.pallas_call(
        body,
        grid=(N // _RB,),
        in_specs=[
            pl.BlockSpec((2, _RB, 128), lambda i: (0, i, 0)),
            pl.BlockSpec((_RB, 128), lambda i: (i, 0)),
            pl.BlockSpec((_RB, 1), lambda i: (i, 0)),
            pl.BlockSpec((128, 256), lambda i: (0, 0)),
            pl.BlockSpec((1, 256), lambda i: (0, 0)),
            pl.BlockSpec((256, 256), lambda i: (0, 0)),
        ],
        out_specs=pl.BlockSpec((2, _RB, 128), lambda i: (0, i, 0)),
        out_shape=jax.ShapeDtypeStruct((2, N, 128), F32),
    )(acc0, v1, dinv_col, w1, b1_2d, w2)


def _tc_dinv(deg2):
    def body(d_ref, o_ref):
        deg = d_ref[0] + d_ref[1] + 1.0
        o_ref[...] = lax.rsqrt(deg)

    return pl.pallas_call(
        body,
        in_specs=[pl.BlockSpec((2, 80, 128), lambda: (0, 0, 0))],
        out_specs=pl.BlockSpec((80, 128), lambda: (0, 0)),
        out_shape=jax.ShapeDtypeStruct((80, 128), F32),
    )(deg2)


def _tc_mid(acc, u, dinv_col, b2d, w, out_c):
    def body(a_ref, u_ref, dv_ref, b_ref, w_ref, o_ref):
        dv = dv_ref[...]
        h0 = jnp.maximum((a_ref[0] + u_ref[0]) * dv + b_ref[:, :128], 0.0)
        h1 = jnp.maximum((a_ref[1] + u_ref[1]) * dv + b_ref[:, 128:], 0.0)
        hw = (jnp.dot(h0, w_ref[:128], preferred_element_type=F32)
              + jnp.dot(h1, w_ref[128:], preferred_element_type=F32))
        if out_c == 256:
            o_ref[0] = hw[:, :128] * dv
            o_ref[1] = hw[:, 128:] * dv
        else:
            o_ref[...] = hw * dv

    if out_c == 256:
        out_spec = pl.BlockSpec((2, _RB, 128), lambda i: (0, i, 0))
        out_shape = jax.ShapeDtypeStruct((2, N, 128), F32)
    else:
        out_spec = pl.BlockSpec((_RB, 128), lambda i: (i, 0))
        out_shape = jax.ShapeDtypeStruct((N, 128), F32)

    return pl.pallas_call(
        body,
        grid=(N // _RB,),
        in_specs=[
            pl.BlockSpec((2, _RB, 128), lambda i: (0, i, 0)),
            pl.BlockSpec((2, _RB, 128), lambda i: (0, i, 0)),
            pl.BlockSpec((_RB, 1), lambda i: (i, 0)),
            pl.BlockSpec((1, 256), lambda i: (0, 0)),
            pl.BlockSpec((256, out_c), lambda i: (0, 0)),
        ],
        out_specs=out_spec,
        out_shape=out_shape,
    )(acc, u, dinv_col, b2d, w)


def _tc_final(acc3, u3, dinv_col, b2d):
    def body(a_ref, u_ref, dv_ref, b_ref, o_ref):
        o_ref[...] = ((a_ref[0] + a_ref[1] + u_ref[...]) * dv_ref[...]
                      + b_ref[...])

    return pl.pallas_call(
        body,
        grid=(N // _RB,),
        in_specs=[
            pl.BlockSpec((2, _RB, 128), lambda i: (0, i, 0)),
            pl.BlockSpec((_RB, 128), lambda i: (i, 0)),
            pl.BlockSpec((_RB, 1), lambda i: (i, 0)),
            pl.BlockSpec((1, 128), lambda i: (0, 0)),
        ],
        out_specs=pl.BlockSpec((_RB, 128), lambda i: (i, 0)),
        out_shape=jax.ShapeDtypeStruct((N, 128), F32),
    )(acc3, u3, dinv_col, b2d)


def _prep_indices(edge_index):
    src = edge_index[0]
    dst = edge_index[1]
    w = jnp.arange(16, dtype=jnp.int32)[:, None]

    pad12 = 480
    padg = (w * 997 + jnp.arange(pad12, dtype=jnp.int32)[None, :] * 13) % N
    pads = N + (w * 31 + jnp.arange(pad12, dtype=jnp.int32)[None, :]) % (N_ACC - N)
    sw = jnp.concatenate([src.reshape(16, E // 16), padg], axis=1)
    dw = jnp.concatenate([dst.reshape(16, E // 16), pads], axis=1)
    gidx12 = jnp.stack([sw, sw + N]).reshape(2, 2560, 128)
    sidx12 = jnp.stack([dw, dw]).reshape(2, 2560, 128)

    pad3 = 240
    padg3 = (w * 997 + jnp.arange(pad3, dtype=jnp.int32)[None, :] * 13) % N
    pads3 = N + (w * 31 + jnp.arange(pad3, dtype=jnp.int32)[None, :]) % (N_ACC - N)
    s3 = src.reshape(2, 16, E // 32)
    d3 = dst.reshape(2, 16, E // 32)
    gidx3 = jnp.concatenate(
        [s3, jnp.broadcast_to(padg3, (2, 16, pad3))], axis=2).reshape(2, 1280, 128)
    sidx3 = jnp.concatenate(
        [d3, jnp.broadcast_to(pads3, (2, 16, pad3))], axis=2).reshape(2, 1280, 128)
    return gidx12, sidx12, gidx3, sidx3


def kernel(x, edge_index, W1, b1, W2, b2, W3, b3):
    gidx12, sidx12, gidx3, sidx3 = _prep_indices(edge_index)

    deg2 = _sc_degree(sidx3)
    dinvp = _tc_dinv(deg2.reshape(2, 80, 128))
    dinv_col = dinvp.reshape(N_ACC, 1)[:N]

    v1 = _tc_scale1(x, dinv_col)
    acc0 = _sc_aggregate(v1, gidx3, sidx3, 80)
    u2 = _tc_l12(acc0, v1, dinv_col, W1, b1.reshape(1, 256), W2)
    acc2 = _sc_aggregate(u2.reshape(2 * N, 128), gidx12, sidx12, 160)
    u3 = _tc_mid(acc2, u2, dinv_col, b2.reshape(1, 256), W3, 128)
    acc3 = _sc_aggregate(u3, gidx3, sidx3, 80)
    return _tc_final(acc3, u3, dinv_col, b3.reshape(1, 128))
